# baseline (speedup 1.0000x reference)
import functools

import jax
import jax.numpy as jnp
from jax import lax
from jax.experimental import pallas as pl
from jax.experimental.pallas import tpu as pltpu
from jax.experimental.pallas import tpu_sc as plsc


def _sc_gather_chunk(table, ids, row_off, rows, d):
    info = plsc.get_sparse_core_info()
    nc, ns = info.num_cores, info.num_subcores
    nw = nc * ns
    per_w = rows // nw
    ch = 128
    n_ch = per_w // ch
    mesh = plsc.VectorSubcoreMesh(core_axis_name="c", subcore_axis_name="s")

    @functools.partial(
        pl.kernel,
        mesh=mesh,
        out_type=jax.ShapeDtypeStruct((rows, d), jnp.float32),
        scratch_types=[
            pltpu.VMEM((per_w,), jnp.int32),
            pltpu.VMEM((per_w, d), jnp.float32),
        ]
        + [pltpu.SemaphoreType.DMA] * (2 * n_ch),
    )
    def gather_k(table_hbm, ids_hbm, out_hbm, idx_v, rows_v, *sems):
        wid = lax.axis_index("s") * nc + lax.axis_index("c")
        base = wid * per_w
        pltpu.sync_copy(ids_hbm.at[pl.ds(row_off + base, per_w)], idx_v)
        gathers = [
            pltpu.async_copy(
                table_hbm.at[idx_v.at[pl.ds(j * ch, ch)]],
                rows_v.at[pl.ds(j * ch, ch)],
                sems[j],
            )
            for j in range(n_ch)
        ]
        writes = []
        for j in range(n_ch):
            gathers[j].wait()
            writes.append(
                pltpu.async_copy(
                    rows_v.at[pl.ds(j * ch, ch)],
                    out_hbm.at[pl.ds(base + j * ch, ch)],
                    sems[n_ch + j],
                )
            )
        for w in writes:
            w.wait()

    return gather_k(table, ids)


def _mlp_body(buf_ref, le_ref, cond_ref, w1_ref, b1_ref, w2_ref, b2_ref,
              w3_ref, b3_ref, out_ref, *, ed):
    del buf_ref
    x = jnp.concatenate([le_ref[:, :ed], cond_ref[...]], axis=1)
    h = jnp.dot(x, w1_ref[...], preferred_element_type=jnp.float32) + b1_ref[...]
    h = jnp.maximum(h, 0.0)
    h = jnp.dot(h, w2_ref[...], preferred_element_type=jnp.float32) + b2_ref[...]
    h = jnp.maximum(h, 0.0)
    logits = jnp.dot(h, w3_ref[...], preferred_element_type=jnp.float32) + b3_ref[...]
    m = jnp.max(logits, axis=1, keepdims=True)
    e = jnp.exp(logits - m)
    out_ref[...] = e / jnp.sum(e, axis=1, keepdims=True)


def _mlp_chunk(buf, le, cond, w1, b1, w2, b2, w3, b3, block_b, off, ed):
    rows = le.shape[0]
    b_full, cd = cond.shape
    out = w3.shape[1]
    grid = (rows // block_b,)
    body = functools.partial(_mlp_body, ed=ed)
    common_specs = [
        pl.BlockSpec((block_b, le.shape[1]), lambda i: (i, 0)),
        pl.BlockSpec((block_b, cd), lambda i, o=off: (i + o, 0)),
        pl.BlockSpec(w1.shape, lambda i: (0, 0)),
        pl.BlockSpec(b1.shape, lambda i: (0, 0)),
        pl.BlockSpec(w2.shape, lambda i: (0, 0)),
        pl.BlockSpec(b2.shape, lambda i: (0, 0)),
        pl.BlockSpec(w3.shape, lambda i: (0, 0)),
        pl.BlockSpec(b3.shape, lambda i: (0, 0)),
    ]
    out_spec = pl.BlockSpec((block_b, out), lambda i, o=off: (i + o, 0))
    out_shape = jax.ShapeDtypeStruct((b_full, out), jnp.float32)
    params = pltpu.CompilerParams(dimension_semantics=("arbitrary",))
    if buf is None:
        def body0(*refs):
            body(None, *refs)
        return pl.pallas_call(
            body0, grid=grid, in_specs=common_specs, out_specs=out_spec,
            out_shape=out_shape, compiler_params=params,
        )(le, cond, w1, b1, w2, b2, w3, b3)
    return pl.pallas_call(
        body, grid=grid,
        in_specs=[pl.BlockSpec(memory_space=pl.ANY)] + common_specs,
        out_specs=out_spec, out_shape=out_shape,
        input_output_aliases={0: 0}, compiler_params=params,
    )(buf, le, cond, w1, b1, w2, b2, w3, b3)


def kernel(conditions, length_ids, emb, W1, b1, W2, b2, W3, b3):
    b = conditions.shape[0]
    d = emb.shape[1]
    block_b = 1024
    chunk_rows = (b // 2, b // 2)
    dpad = 128
    emb_p = jnp.pad(emb, ((0, 0), (0, dpad - d)))
    ids = length_ids.astype(jnp.int32)
    b1r, b2r, b3r = b1.reshape(1, -1), b2.reshape(1, -1), b3.reshape(1, -1)

    n_chunk = len(chunk_rows)
    offs = [sum(chunk_rows[:c]) for c in range(n_chunk + 1)]
    les = [None] * n_chunk
    les[0] = _sc_gather_chunk(emb_p, ids, offs[0], chunk_rows[0], dpad)
    buf = None
    for c in range(n_chunk):
        if c + 1 < n_chunk:
            les[c + 1] = _sc_gather_chunk(emb_p, ids, offs[c + 1],
                                          chunk_rows[c + 1], dpad)
        buf = _mlp_chunk(buf, les[c], conditions, W1, b1r, W2, b2r, W3, b3r,
                         block_b=block_b, off=offs[c] // block_b, ed=d)
    return buf


# speedup vs baseline: 1.0663x; 1.0663x over previous
import functools

import jax
import jax.numpy as jnp
from jax import lax
from jax.experimental import pallas as pl
from jax.experimental.pallas import tpu as pltpu
from jax.experimental.pallas import tpu_sc as plsc


def _sc_gather_chunk(table, ids, row_off, rows, d):
    info = plsc.get_sparse_core_info()
    nc, ns = info.num_cores, info.num_subcores
    nw = nc * ns
    per_w = rows // nw
    ch = 64
    n_ch = per_w // ch
    mesh = plsc.VectorSubcoreMesh(core_axis_name="c", subcore_axis_name="s")

    @functools.partial(
        pl.kernel,
        mesh=mesh,
        out_type=jax.ShapeDtypeStruct((rows, d), jnp.float32),
        scratch_types=[
            pltpu.VMEM((per_w,), jnp.int32),
            pltpu.VMEM((per_w, d), jnp.float32),
        ]
        + [pltpu.SemaphoreType.DMA] * (2 * n_ch),
    )
    def gather_k(table_hbm, ids_hbm, out_hbm, idx_v, rows_v, *sems):
        wid = lax.axis_index("s") * nc + lax.axis_index("c")
        base = wid * per_w
        pltpu.sync_copy(ids_hbm.at[pl.ds(row_off + base, per_w)], idx_v)
        gathers = [
            pltpu.async_copy(
                table_hbm.at[idx_v.at[pl.ds(j * ch, ch)]],
                rows_v.at[pl.ds(j * ch, ch)],
                sems[j],
            )
            for j in range(n_ch)
        ]
        writes = []
        for j in range(n_ch):
            gathers[j].wait()
            writes.append(
                pltpu.async_copy(
                    rows_v.at[pl.ds(j * ch, ch)],
                    out_hbm.at[pl.ds(base + j * ch, ch)],
                    sems[n_ch + j],
                )
            )
        for w in writes:
            w.wait()

    return gather_k(table, ids)


def _mlp_body(buf_ref, le_ref, cond_ref, w1_ref, b1_ref, w2_ref, b2_ref,
              w3_ref, b3_ref, out_ref, *, ed):
    del buf_ref
    x = jnp.concatenate([le_ref[:, :ed], cond_ref[...]], axis=1)
    h = jnp.dot(x, w1_ref[...], preferred_element_type=jnp.float32) + b1_ref[...]
    h = jnp.maximum(h, 0.0)
    h = jnp.dot(h, w2_ref[...], preferred_element_type=jnp.float32) + b2_ref[...]
    h = jnp.maximum(h, 0.0)
    logits = jnp.dot(h, w3_ref[...], preferred_element_type=jnp.float32) + b3_ref[...]
    m = jnp.max(logits, axis=1, keepdims=True)
    e = jnp.exp(logits - m)
    out_ref[...] = e / jnp.sum(e, axis=1, keepdims=True)


def _mlp_chunk(buf, le, cond, w1, b1, w2, b2, w3, b3, block_b, off, ed):
    rows = le.shape[0]
    b_full, cd = cond.shape
    out = w3.shape[1]
    grid = (rows // block_b,)
    body = functools.partial(_mlp_body, ed=ed)
    common_specs = [
        pl.BlockSpec((block_b, le.shape[1]), lambda i: (i, 0)),
        pl.BlockSpec((block_b, cd), lambda i, o=off: (i + o, 0)),
        pl.BlockSpec(w1.shape, lambda i: (0, 0)),
        pl.BlockSpec(b1.shape, lambda i: (0, 0)),
        pl.BlockSpec(w2.shape, lambda i: (0, 0)),
        pl.BlockSpec(b2.shape, lambda i: (0, 0)),
        pl.BlockSpec(w3.shape, lambda i: (0, 0)),
        pl.BlockSpec(b3.shape, lambda i: (0, 0)),
    ]
    out_spec = pl.BlockSpec((block_b, out), lambda i, o=off: (i + o, 0))
    out_shape = jax.ShapeDtypeStruct((b_full, out), jnp.float32)
    params = pltpu.CompilerParams(dimension_semantics=("arbitrary",))
    if buf is None:
        def body0(*refs):
            body(None, *refs)
        return pl.pallas_call(
            body0, grid=grid, in_specs=common_specs, out_specs=out_spec,
            out_shape=out_shape, compiler_params=params,
        )(le, cond, w1, b1, w2, b2, w3, b3)
    return pl.pallas_call(
        body, grid=grid,
        in_specs=[pl.BlockSpec(memory_space=pl.ANY)] + common_specs,
        out_specs=out_spec, out_shape=out_shape,
        input_output_aliases={0: 0}, compiler_params=params,
    )(buf, le, cond, w1, b1, w2, b2, w3, b3)


def kernel(conditions, length_ids, emb, W1, b1, W2, b2, W3, b3):
    b = conditions.shape[0]
    d = emb.shape[1]
    block_b = 4096
    chunk_rows = (b // 2, b // 2)
    dpad = 128
    emb_p = jnp.pad(emb, ((0, 0), (0, dpad - d)))
    ids = length_ids.astype(jnp.int32)
    b1r, b2r, b3r = b1.reshape(1, -1), b2.reshape(1, -1), b3.reshape(1, -1)

    n_chunk = len(chunk_rows)
    offs = [sum(chunk_rows[:c]) for c in range(n_chunk + 1)]
    les = [None] * n_chunk
    les[0] = _sc_gather_chunk(emb_p, ids, offs[0], chunk_rows[0], dpad)
    buf = None
    for c in range(n_chunk):
        if c + 1 < n_chunk:
            les[c + 1] = _sc_gather_chunk(emb_p, ids, offs[c + 1],
                                          chunk_rows[c + 1], dpad)
        buf = _mlp_chunk(buf, les[c], conditions, W1, b1r, W2, b2r, W3, b3r,
                         block_b=block_b, off=offs[c] // block_b, ed=d)
    return buf


# speedup vs baseline: 1.0677x; 1.0014x over previous
import functools

import jax
import jax.numpy as jnp
from jax import lax
from jax.experimental import pallas as pl
from jax.experimental.pallas import tpu as pltpu
from jax.experimental.pallas import tpu_sc as plsc


def _sc_gather_chunk(table, ids, row_off, rows, d):
    info = plsc.get_sparse_core_info()
    nc, ns = info.num_cores, info.num_subcores
    nw = nc * ns
    per_w = rows // nw
    ch = 128
    n_ch = per_w // ch
    mesh = plsc.VectorSubcoreMesh(core_axis_name="c", subcore_axis_name="s")

    @functools.partial(
        pl.kernel,
        mesh=mesh,
        out_type=jax.ShapeDtypeStruct((rows, d), jnp.float32),
        scratch_types=[
            pltpu.VMEM((per_w,), jnp.int32),
            pltpu.VMEM((per_w, d), jnp.float32),
        ]
        + [pltpu.SemaphoreType.DMA] * (2 * n_ch),
    )
    def gather_k(table_hbm, ids_hbm, out_hbm, idx_v, rows_v, *sems):
        wid = lax.axis_index("s") * nc + lax.axis_index("c")
        base = wid * per_w
        pltpu.sync_copy(ids_hbm.at[pl.ds(row_off + base, per_w)], idx_v)
        gathers = [
            pltpu.async_copy(
                table_hbm.at[idx_v.at[pl.ds(j * ch, ch)]],
                rows_v.at[pl.ds(j * ch, ch)],
                sems[j],
            )
            for j in range(n_ch)
        ]
        writes = []
        for j in range(n_ch):
            gathers[j].wait()
            writes.append(
                pltpu.async_copy(
                    rows_v.at[pl.ds(j * ch, ch)],
                    out_hbm.at[pl.ds(base + j * ch, ch)],
                    sems[n_ch + j],
                )
            )
        for w in writes:
            w.wait()

    return gather_k(table, ids)


def _mlp_body(buf_ref, le_ref, cond_ref, w1_ref, b1_ref, w2_ref, b2_ref,
              w3_ref, b3_ref, out_ref, *, ed):
    del buf_ref
    x = jnp.concatenate([le_ref[:, :ed], cond_ref[...]], axis=1)
    h = jnp.dot(x, w1_ref[...], preferred_element_type=jnp.float32) + b1_ref[...]
    h = jnp.maximum(h, 0.0)
    h = jnp.dot(h, w2_ref[...], preferred_element_type=jnp.float32) + b2_ref[...]
    h = jnp.maximum(h, 0.0)
    logits = jnp.dot(h, w3_ref[...], preferred_element_type=jnp.float32) + b3_ref[...]
    m = jnp.max(logits, axis=1, keepdims=True)
    e = jnp.exp(logits - m)
    out_ref[...] = e / jnp.sum(e, axis=1, keepdims=True)


def _mlp_chunk(buf, le, cond, w1, b1, w2, b2, w3, b3, block_b, off, ed):
    rows = le.shape[0]
    b_full, cd = cond.shape
    out = w3.shape[1]
    grid = (rows // block_b,)
    body = functools.partial(_mlp_body, ed=ed)
    common_specs = [
        pl.BlockSpec((block_b, le.shape[1]), lambda i: (i, 0)),
        pl.BlockSpec((block_b, cd), lambda i, o=off: (i + o, 0)),
        pl.BlockSpec(w1.shape, lambda i: (0, 0)),
        pl.BlockSpec(b1.shape, lambda i: (0, 0)),
        pl.BlockSpec(w2.shape, lambda i: (0, 0)),
        pl.BlockSpec(b2.shape, lambda i: (0, 0)),
        pl.BlockSpec(w3.shape, lambda i: (0, 0)),
        pl.BlockSpec(b3.shape, lambda i: (0, 0)),
    ]
    out_spec = pl.BlockSpec((block_b, out), lambda i, o=off: (i + o, 0))
    out_shape = jax.ShapeDtypeStruct((b_full, out), jnp.float32)
    params = pltpu.CompilerParams(dimension_semantics=("arbitrary",))
    if buf is None:
        def body0(*refs):
            body(None, *refs)
        return pl.pallas_call(
            body0, grid=grid, in_specs=common_specs, out_specs=out_spec,
            out_shape=out_shape, compiler_params=params,
        )(le, cond, w1, b1, w2, b2, w3, b3)
    return pl.pallas_call(
        body, grid=grid,
        in_specs=[pl.BlockSpec(memory_space=pl.ANY)] + common_specs,
        out_specs=out_spec, out_shape=out_shape,
        input_output_aliases={0: 0}, compiler_params=params,
    )(buf, le, cond, w1, b1, w2, b2, w3, b3)


def kernel(conditions, length_ids, emb, W1, b1, W2, b2, W3, b3):
    b = conditions.shape[0]
    d = emb.shape[1]
    block_b = 4096
    chunk_rows = (b // 2, b // 2)
    dpad = 128
    emb_p = jnp.pad(emb, ((0, 0), (0, dpad - d)))
    ids = length_ids.astype(jnp.int32)
    b1r, b2r, b3r = b1.reshape(1, -1), b2.reshape(1, -1), b3.reshape(1, -1)

    n_chunk = len(chunk_rows)
    offs = [sum(chunk_rows[:c]) for c in range(n_chunk + 1)]
    les = [None] * n_chunk
    les[0] = _sc_gather_chunk(emb_p, ids, offs[0], chunk_rows[0], dpad)
    buf = None
    for c in range(n_chunk):
        if c + 1 < n_chunk:
            les[c + 1] = _sc_gather_chunk(emb_p, ids, offs[c + 1],
                                          chunk_rows[c + 1], dpad)
        buf = _mlp_chunk(buf, les[c], conditions, W1, b1r, W2, b2r, W3, b3r,
                         block_b=block_b, off=offs[c] // block_b, ed=d)
    return buf


# speedup vs baseline: 1.0917x; 1.0225x over previous
import functools

import jax
import jax.numpy as jnp
from jax import lax
from jax.experimental import pallas as pl
from jax.experimental.pallas import tpu as pltpu
from jax.experimental.pallas import tpu_sc as plsc


def _sc_gather_chunk(table, ids, row_off, rows, d):
    info = plsc.get_sparse_core_info()
    nc, ns = info.num_cores, info.num_subcores
    nw = nc * ns
    per_w = rows // nw
    ch = 128
    n_ch = per_w // ch
    mesh = plsc.VectorSubcoreMesh(core_axis_name="c", subcore_axis_name="s")

    @functools.partial(
        pl.kernel,
        mesh=mesh,
        out_type=jax.ShapeDtypeStruct((rows, d), jnp.float32),
        scratch_types=[
            pltpu.VMEM((per_w,), jnp.int32),
            pltpu.VMEM((per_w, d), jnp.float32),
        ]
        + [pltpu.SemaphoreType.DMA] * (2 * n_ch),
    )
    def gather_k(table_hbm, ids_hbm, out_hbm, idx_v, rows_v, *sems):
        wid = lax.axis_index("s") * nc + lax.axis_index("c")
        base = wid * per_w
        pltpu.sync_copy(ids_hbm.at[pl.ds(row_off + base, per_w)], idx_v)
        gathers = [
            pltpu.async_copy(
                table_hbm.at[idx_v.at[pl.ds(j * ch, ch)]],
                rows_v.at[pl.ds(j * ch, ch)],
                sems[j],
            )
            for j in range(n_ch)
        ]
        writes = []
        for j in range(n_ch):
            gathers[j].wait()
            writes.append(
                pltpu.async_copy(
                    rows_v.at[pl.ds(j * ch, ch)],
                    out_hbm.at[pl.ds(base + j * ch, ch)],
                    sems[n_ch + j],
                )
            )
        for w in writes:
            w.wait()

    return gather_k(table, ids)


def _mlp_body(buf_ref, le_ref, cond_ref, w1_ref, b1_ref, w2_ref, b2_ref,
              w3_ref, b3_ref, out_ref, *, ed):
    del buf_ref
    x = jnp.concatenate([le_ref[:, :ed], cond_ref[...]], axis=1)
    h = jnp.dot(x, w1_ref[...], preferred_element_type=jnp.float32) + b1_ref[...]
    h = jnp.maximum(h, 0.0)
    h = jnp.dot(h, w2_ref[...], preferred_element_type=jnp.float32) + b2_ref[...]
    h = jnp.maximum(h, 0.0)
    logits = jnp.dot(h, w3_ref[...], preferred_element_type=jnp.float32) + b3_ref[...]
    m = jnp.max(logits, axis=1, keepdims=True)
    e = jnp.exp(logits - m)
    out_ref[...] = e / jnp.sum(e, axis=1, keepdims=True)


def _mlp_chunk(buf, le, cond, w1, b1, w2, b2, w3, b3, block_b, off, ed):
    rows = le.shape[0]
    b_full, cd = cond.shape
    out = w3.shape[1]
    grid = (rows // block_b,)
    body = functools.partial(_mlp_body, ed=ed)
    common_specs = [
        pl.BlockSpec((block_b, le.shape[1]), lambda i: (i, 0)),
        pl.BlockSpec((block_b, cd), lambda i, o=off: (i + o, 0)),
        pl.BlockSpec(w1.shape, lambda i: (0, 0)),
        pl.BlockSpec(b1.shape, lambda i: (0, 0)),
        pl.BlockSpec(w2.shape, lambda i: (0, 0)),
        pl.BlockSpec(b2.shape, lambda i: (0, 0)),
        pl.BlockSpec(w3.shape, lambda i: (0, 0)),
        pl.BlockSpec(b3.shape, lambda i: (0, 0)),
    ]
    out_spec = pl.BlockSpec((block_b, out), lambda i, o=off: (i + o, 0))
    out_shape = jax.ShapeDtypeStruct((b_full, out), jnp.float32)
    params = pltpu.CompilerParams(dimension_semantics=("arbitrary",))
    if buf is None:
        def body0(*refs):
            body(None, *refs)
        return pl.pallas_call(
            body0, grid=grid, in_specs=common_specs, out_specs=out_spec,
            out_shape=out_shape, compiler_params=params,
        )(le, cond, w1, b1, w2, b2, w3, b3)
    return pl.pallas_call(
        body, grid=grid,
        in_specs=[pl.BlockSpec(memory_space=pl.ANY)] + common_specs,
        out_specs=out_spec, out_shape=out_shape,
        input_output_aliases={0: 0}, compiler_params=params,
    )(buf, le, cond, w1, b1, w2, b2, w3, b3)


def kernel(conditions, length_ids, emb, W1, b1, W2, b2, W3, b3):
    b = conditions.shape[0]
    d = emb.shape[1]
    block_b = 2048
    chunk_rows = (3 * b // 8, 5 * b // 8)
    dpad = 128
    emb_p = jnp.pad(emb, ((0, 0), (0, dpad - d)))
    ids = length_ids.astype(jnp.int32)
    b1r, b2r, b3r = b1.reshape(1, -1), b2.reshape(1, -1), b3.reshape(1, -1)

    n_chunk = len(chunk_rows)
    offs = [sum(chunk_rows[:c]) for c in range(n_chunk + 1)]
    les = [None] * n_chunk
    les[0] = _sc_gather_chunk(emb_p, ids, offs[0], chunk_rows[0], dpad)
    buf = None
    for c in range(n_chunk):
        if c + 1 < n_chunk:
            les[c + 1] = _sc_gather_chunk(emb_p, ids, offs[c + 1],
                                          chunk_rows[c + 1], dpad)
        buf = _mlp_chunk(buf, les[c], conditions, W1, b1r, W2, b2r, W3, b3r,
                         block_b=block_b, off=offs[c] // block_b, ed=d)
    return buf
